# Initial kernel scaffold; baseline (speedup 1.0000x reference)
import jax
import jax.numpy as jnp
from jax.experimental import pallas as pl


def kernel(grid_nfeat, mesh_nfeat, mesh_efeat, g2m_efeat, m2g_efeat, mesh_edge_index, g2m_edge_index, m2g_edge_index, params):
    raise NotImplementedError("write your pallas kernel here")


# speedup vs baseline: 1.6623x; 1.6623x over previous
import functools

import jax
import jax.numpy as jnp
from jax import lax
from jax.experimental import pallas as pl
from jax.experimental.pallas import tpu as pltpu
from jax.experimental.pallas import tpu_sc as plsc

N_GRID = 50000
N_MESH = 10000
E_MESH = 80000
E_G2M = 80000
E_M2G = 150000
HID = 128

NW = 32
CHUNK = 128

EP_G2M = 81920
EP_MESH = 81920
EP_M2G = 151552

MESH_ACC = 10240
MESH_DUMP = 10016
GRID_RANGE = 12800
GRID_ACC = 12928
GRID_DUMP = 12800
GRID_OUT = 51200


def _tc_mlp(xs, w0s, b0, w1, b1, gb=None, res=None, chain=None, blk=1024,
            rows=None):
    nx = len(xs)
    dout = (chain[2] if chain else w1).shape[1]
    n_blocks = rows // blk

    def body(*refs):
        it = iter(refs)
        x_refs = [next(it) for _ in range(nx)]
        w_refs = [next(it) for _ in range(nx)]
        b0r, w1r, b1r = next(it), next(it), next(it)
        gr = br = rr = None
        if gb is not None:
            gr, br = next(it), next(it)
        if res is not None:
            rr = next(it)
        c_refs = [next(it) for _ in range(4)] if chain else None
        out = next(it)

        acc = b0r[...].astype(jnp.float32)
        for xr, wr in zip(x_refs, w_refs):
            acc = acc + jnp.dot(xr[...], wr[...],
                                preferred_element_type=jnp.float32)
        h = acc * jax.nn.sigmoid(acc)
        y = jnp.dot(h, w1r[...], preferred_element_type=jnp.float32) + b1r[...]
        if gb is not None:
            mu = jnp.mean(y, axis=-1, keepdims=True)
            var = jnp.mean((y - mu) ** 2, axis=-1, keepdims=True)
            y = (y - mu) * lax.rsqrt(var + 1e-5) * gr[...] + br[...]
        if res is not None:
            y = y + rr[...]
        if chain:
            cw0, cb0, cw1, cb1 = (r[...] for r in c_refs)
            h2 = jnp.dot(y, cw0, preferred_element_type=jnp.float32) + cb0
            h2 = h2 * jax.nn.sigmoid(h2)
            y = jnp.dot(h2, cw1, preferred_element_type=jnp.float32) + cb1
        out[...] = y

    def row_spec(arr):
        d = arr.shape[1]
        nb = -(-arr.shape[0] // blk)
        return pl.BlockSpec((blk, d), lambda i, nb=nb: (jnp.minimum(i, nb - 1), 0))

    def full_spec(arr):
        return pl.BlockSpec(arr.shape, lambda i: (0,) * arr.ndim)

    operands = list(xs) + list(w0s) + [b0, w1, b1]
    specs = [row_spec(x) for x in xs] + [full_spec(w) for w in w0s] + \
            [full_spec(b0), full_spec(w1), full_spec(b1)]
    if gb is not None:
        operands += [gb[0], gb[1]]
        specs += [full_spec(gb[0]), full_spec(gb[1])]
    if res is not None:
        operands.append(res)
        specs.append(row_spec(res))
    if chain:
        operands += list(chain)
        specs += [full_spec(c) for c in chain]

    return pl.pallas_call(
        body,
        grid=(n_blocks,),
        in_specs=specs,
        out_specs=pl.BlockSpec((blk, dout), lambda i: (i, 0)),
        out_shape=jax.ShapeDtypeStruct((rows, dout), jnp.float32),
    )(*operands)


def _mlp_p(p, x, rows, blk=1024, res=None):
    gb = (p["g"].reshape(1, -1), p["beta"].reshape(1, -1)) if "g" in p else None
    return _tc_mlp([x], [p["W0"]], p["b0"].reshape(1, -1), p["W1"],
                   p["b1"].reshape(1, -1), gb=gb, res=res, blk=blk, rows=rows)


def _mlp_cat(p, xs, rows, blk=1024, res=None, dup_first=False, chain=None):
    d0 = xs[0].shape[1] if not dup_first else HID
    pieces = []
    off = 0
    sizes = []
    if dup_first:
        sizes = [HID, HID]
        pieces = [p["W0"][:HID], p["W0"][:HID]]
        off = HID
        rest = xs[2:]
    else:
        rest = xs
    for x in rest:
        d = x.shape[1]
        pieces.append(p["W0"][off:off + d])
        off += d
    gb = (p["g"].reshape(1, -1), p["beta"].reshape(1, -1)) if "g" in p else None
    return _tc_mlp(xs, pieces, p["b0"].reshape(1, -1), p["W1"],
                   p["b1"].reshape(1, -1), gb=gb, res=res, chain=chain,
                   blk=blk, rows=rows)


@functools.cache
def _sc_mesh():
    return plsc.VectorSubcoreMesh(core_axis_name="c", subcore_axis_name="s")


def _sc_gather2(table_a, idx_a, table_b, idx_b):
    e = idx_a.shape[0]
    ch = e // (NW * CHUNK)

    @functools.partial(
        pl.kernel,
        out_type=(jax.ShapeDtypeStruct((e, HID), jnp.float32),
                  jax.ShapeDtypeStruct((e, HID), jnp.float32)),
        mesh=_sc_mesh(),
        scratch_types=[
            pltpu.VMEM((CHUNK,), jnp.int32),
            pltpu.VMEM((CHUNK, HID), jnp.float32),
            pltpu.VMEM((CHUNK,), jnp.int32),
            pltpu.VMEM((CHUNK, HID), jnp.float32),
            pltpu.SemaphoreType.DMA,
            pltpu.SemaphoreType.DMA,
        ],
    )
    def k(ta, ia, tb, ib, oa, ob, iva, rva, ivb, rvb, sema, semb):
        wid = lax.axis_index("s") * 2 + lax.axis_index("c")
        base = wid * ch

        def body(j, carry):
            off = (base + j) * CHUNK
            pltpu.sync_copy(ia.at[pl.ds(off, CHUNK)], iva)
            cpa = pltpu.async_copy(ta.at[iva], rva, sema)
            pltpu.sync_copy(ib.at[pl.ds(off, CHUNK)], ivb)
            cpb = pltpu.async_copy(tb.at[ivb], rvb, semb)
            cpa.wait()
            pltpu.sync_copy(rva, oa.at[pl.ds(off, CHUNK)])
            cpb.wait()
            pltpu.sync_copy(rvb, ob.at[pl.ds(off, CHUNK)])
            return carry

        lax.fori_loop(0, ch, body, 0)

    return k(table_a, idx_a, table_b, idx_b)


def _sc_scatter_mesh(values, idx, zeros):
    e = values.shape[0]
    ch = e // (NW * CHUNK)
    pt = MESH_ACC // 16

    @functools.partial(
        pl.kernel,
        out_type=(jax.ShapeDtypeStruct((MESH_ACC, HID), jnp.float32),
                  jax.ShapeDtypeStruct((MESH_ACC, HID), jnp.float32)),
        mesh=_sc_mesh(),
        scratch_types=[
            pltpu.VMEM((CHUNK,), jnp.int32),
            pltpu.VMEM((CHUNK, HID), jnp.float32),
            pltpu.VMEM_SHARED((MESH_ACC, HID), jnp.float32),
        ],
    )
    def k(vals, ix, zr, o0, o1, idxv, rows, acc):
        c = lax.axis_index("c")
        s = lax.axis_index("s")
        wid = s * 2 + c
        pltpu.sync_copy(zr.at[pl.ds(s * pt, pt)], acc.at[pl.ds(s * pt, pt)])
        plsc.subcore_barrier()

        def body(j, carry):
            off = (wid * ch + j) * CHUNK
            pltpu.sync_copy(ix.at[pl.ds(off, CHUNK)], idxv)
            pltpu.sync_copy(vals.at[pl.ds(off, CHUNK)], rows)
            pltpu.sync_copy(rows, acc.at[idxv], add=True)
            return carry

        lax.fori_loop(0, ch, body, 0)
        plsc.subcore_barrier()

        @pl.when(c == 0)
        def _():
            pltpu.sync_copy(acc.at[pl.ds(s * pt, pt)], o0.at[pl.ds(s * pt, pt)])

        @pl.when(c == 1)
        def _():
            pltpu.sync_copy(acc.at[pl.ds(s * pt, pt)], o1.at[pl.ds(s * pt, pt)])

    return k(values, idx, zeros)


def _sc_scatter_grid(values, idx, zeros):
    e = values.shape[0]
    ch = e // (NW * CHUNK)
    zt = GRID_ACC // 16
    ot = GRID_RANGE // 16

    @functools.partial(
        pl.kernel,
        out_type=(jax.ShapeDtypeStruct((GRID_OUT, HID), jnp.float32),
                  jax.ShapeDtypeStruct((GRID_OUT, HID), jnp.float32)),
        mesh=_sc_mesh(),
        scratch_types=[
            pltpu.VMEM((CHUNK,), jnp.int32),
            pltpu.VMEM((CHUNK,), jnp.int32),
            pltpu.VMEM((CHUNK, HID), jnp.float32),
            pltpu.VMEM_SHARED((GRID_ACC, HID), jnp.float32),
        ],
    )
    def k(vals, ix, zr, o0, o1, idxv, idxt, rows, acc):
        c = lax.axis_index("c")
        s = lax.axis_index("s")
        wid = s * 2 + c

        def range_pass(r, carry):
            lo = r * GRID_RANGE
            hi = jnp.minimum(lo + GRID_RANGE, N_GRID)
            pltpu.sync_copy(zr.at[pl.ds(s * zt, zt)], acc.at[pl.ds(s * zt, zt)])
            plsc.subcore_barrier()

            def body(j, carry2):
                off = (wid * ch + j) * CHUNK
                pltpu.sync_copy(ix.at[pl.ds(off, CHUNK)], idxv)
                for kk in range(CHUNK // 16):
                    v = idxv[pl.ds(kk * 16, 16)]
                    inr = (v >= lo) & (v < hi)
                    idxt[pl.ds(kk * 16, 16)] = jnp.where(inr, v - lo, GRID_DUMP)
                pltpu.sync_copy(vals.at[pl.ds(off, CHUNK)], rows)
                pltpu.sync_copy(rows, acc.at[idxt], add=True)
                return carry2

            lax.fori_loop(0, ch, body, 0)
            plsc.subcore_barrier()

            @pl.when(c == 0)
            def _():
                pltpu.sync_copy(acc.at[pl.ds(s * ot, ot)],
                                o0.at[pl.ds(lo + s * ot, ot)])

            @pl.when(c == 1)
            def _():
                pltpu.sync_copy(acc.at[pl.ds(s * ot, ot)],
                                o1.at[pl.ds(lo + s * ot, ot)])

            plsc.subcore_barrier()
            return carry

        lax.fori_loop(0, 4, range_pass, 0)

    return k(values, idx, zeros)


def _pad_idx(idx, e_pad, fill):
    return jnp.concatenate(
        [idx, jnp.full((e_pad - idx.shape[0],), fill, jnp.int32)])


def kernel(grid_nfeat, mesh_nfeat, mesh_efeat, g2m_efeat, m2g_efeat,
           mesh_edge_index, g2m_edge_index, m2g_edge_index, params):
    p = params
    zeros = jnp.zeros((GRID_ACC, HID), jnp.float32)

    grid_h = _mlp_p(p["grid_embed"], grid_nfeat, rows=N_GRID, blk=1000)
    mesh_h = _mlp_p(p["mesh_embed"], mesh_nfeat, rows=N_MESH, blk=1000)
    g2m_eh = _mlp_p(p["g2m_edge_embed"], g2m_efeat, rows=EP_G2M)
    mesh_eh = _mlp_p(p["mesh_edge_embed"], mesh_efeat, rows=EP_MESH)

    src = _pad_idx(g2m_edge_index[0], EP_G2M, 0)
    dst_g = _pad_idx(g2m_edge_index[1], EP_G2M, 0)
    dst_s = _pad_idx(g2m_edge_index[1], EP_G2M, MESH_DUMP)
    gs, gd = _sc_gather2(grid_h, src, mesh_h, dst_g)
    e1 = _mlp_cat(p["enc_edge"], [g2m_eh, gs, gd], rows=EP_G2M, res=g2m_eh)
    a0, a1 = _sc_scatter_mesh(e1, dst_s, zeros)
    mesh_h = _mlp_cat(p["enc_dst_node"], [a0, a1, mesh_h], rows=N_MESH,
                      blk=1000, res=mesh_h, dup_first=True)
    grid_h = _mlp_p(p["enc_src_node"], grid_h, rows=N_GRID, blk=1000,
                    res=grid_h)

    sm = _pad_idx(mesh_edge_index[0], EP_MESH, 0)
    dm_g = _pad_idx(mesh_edge_index[1], EP_MESH, 0)
    dm_s = _pad_idx(mesh_edge_index[1], EP_MESH, MESH_DUMP)
    e, n = mesh_eh, mesh_h
    for layer in p["proc"]:
        gs, gd = _sc_gather2(n, sm, n, dm_g)
        e = _mlp_cat(layer["edge"], [e, gs, gd], rows=EP_MESH, res=e)
        a0, a1 = _sc_scatter_mesh(e, dm_s, zeros)
        n = _mlp_cat(layer["node"], [a0, a1, n], rows=N_MESH, blk=1000,
                     res=n, dup_first=True)
    mesh_h = n

    m2g_eh = _mlp_p(p["m2g_edge_embed"], m2g_efeat, rows=EP_M2G)
    sd = _pad_idx(m2g_edge_index[0], EP_M2G, 0)
    dd_g = _pad_idx(m2g_edge_index[1], EP_M2G, 0)
    dd_s = _pad_idx(m2g_edge_index[1], EP_M2G, GRID_OUT)
    gs, gd = _sc_gather2(mesh_h, sd, grid_h, dd_g)
    e2 = _mlp_cat(p["dec_edge"], [m2g_eh, gs, gd], rows=EP_M2G, res=m2g_eh)
    q0, q1 = _sc_scatter_grid(e2, dd_s, zeros)

    fin = p["finale"]
    out = _mlp_cat(p["dec_node"], [q0, q1, grid_h], rows=N_GRID, blk=1000,
                   res=grid_h, dup_first=True,
                   chain=(fin["W0"], fin["b0"].reshape(1, -1),
                          fin["W1"], fin["b1"].reshape(1, -1)))
    return out
